# baseline (speedup 1.0000x reference)
import functools

import jax
import jax.numpy as jnp
from jax import lax
from jax.experimental import pallas as pl
from jax.experimental.pallas import tpu as pltpu
from jax.experimental.pallas import tpu_sc as plsc

IN_F = 2048
OUT_F = 2048
NCONN = 16

NUM_CORES = 2
NUM_SUBCORES = 16
NW = NUM_CORES * NUM_SUBCORES
ROWS_PER_W = OUT_F // NW
SUB = 2
NCHUNK = ROWS_PER_W // SUB
NBUF = 4


def _build_mt(conn_i32, weight, zeros_blk):
    mesh = plsc.VectorSubcoreMesh(core_axis_name="c", subcore_axis_name="s")

    @functools.partial(
        pl.kernel,
        mesh=mesh,
        out_type=jax.ShapeDtypeStruct((OUT_F, IN_F), jnp.float32),
        scratch_types=(
            [pltpu.VMEM((SUB, IN_F), jnp.float32) for _ in range(NBUF)]
            + [pltpu.VMEM((ROWS_PER_W * NCONN,), jnp.int32),
               pltpu.VMEM((ROWS_PER_W, NCONN), jnp.float32)]
            + [pltpu.SemaphoreType.DMA for _ in range(2 * NBUF)]
        ),
        compiler_params=pltpu.CompilerParams(needs_layout_passes=False),
    )
    def k(conn_hbm, w_hbm, z_hbm, mt_hbm, *refs):
        blks = refs[:NBUF]
        idx_v, wv = refs[NBUF], refs[NBUF + 1]
        sems = refs[NBUF + 2:NBUF + 2 + NBUF]
        zsems = refs[NBUF + 2 + NBUF:]
        wid = lax.axis_index("s") * NUM_CORES + lax.axis_index("c")
        base = wid * ROWS_PER_W
        zc = [pltpu.async_copy(z_hbm, blks[b], zsems[b])
              for b in range(NBUF)]
        pltpu.sync_copy(conn_hbm.at[pl.ds(base * NCONN, ROWS_PER_W * NCONN)],
                        idx_v)
        pltpu.sync_copy(w_hbm.at[pl.ds(base, ROWS_PER_W)], wv)
        for h in zc:
            h.wait()
        zvec = jnp.zeros((NCONN,), jnp.float32)

        def _chunk(g, _):
            for b in range(NBUF):
                c = g * NBUF + b
                blk = blks[b]

                @pl.when(g > 0)
                def _wait_and_restore(blk=blk, c=c, b=b):
                    pltpu.make_async_copy(
                        blk, mt_hbm.at[pl.ds(base + (c - NBUF) * SUB, SUB)],
                        sems[b]).wait()

                    def _restore(r, _):
                        o = ((c - NBUF) * SUB + r) * NCONN
                        rvec = jnp.full((NCONN,), r, jnp.int32)
                        plsc.store_scatter(blk,
                                           [rvec, idx_v[pl.ds(o, NCONN)]],
                                           zvec)
                        return _

                    lax.fori_loop(0, SUB, _restore, None, unroll=SUB)

                def _scatter(r, _, blk=blk, c=c):
                    o = (c * SUB + r) * NCONN
                    idx = idx_v[pl.ds(o, NCONN)]
                    w = wv[c * SUB + r]
                    rvec = jnp.full((NCONN,), r, jnp.int32)
                    plsc.addupdate_scatter(blk, [rvec, idx], w)
                    return _

                lax.fori_loop(0, SUB, _scatter, None, unroll=SUB)
                pltpu.async_copy(
                    blk, mt_hbm.at[pl.ds(base + c * SUB, SUB)], sems[b])
            return _

        lax.fori_loop(0, NCHUNK // NBUF, _chunk, None)
        for b in range(NBUF):
            c_last = NCHUNK - NBUF + b
            pltpu.make_async_copy(
                blks[b], mt_hbm.at[pl.ds(base + c_last * SUB, SUB)],
                sems[b]).wait()

    return k(conn_i32, weight, zeros_blk)


def _matmul(x2d, mt, bias):
    A_BLK = 256

    def body(x_ref, mt_ref, b_ref, o_ref):
        acc = lax.dot_general(
            x_ref[...], mt_ref[...],
            (((1,), (1,)), ((), ())),
            preferred_element_type=jnp.float32,
        )
        o_ref[...] = (acc + b_ref[...])[None]

    return pl.pallas_call(
        body,
        grid=(OUT_F // A_BLK,),
        in_specs=[
            pl.BlockSpec((2048, IN_F), lambda i: (0, 0)),
            pl.BlockSpec((A_BLK, IN_F), lambda i: (i, 0)),
            pl.BlockSpec((A_BLK,), lambda i: (i,)),
        ],
        out_specs=pl.BlockSpec((1, 2048, A_BLK), lambda i: (0, 0, i)),
        out_shape=jax.ShapeDtypeStruct((1, 2048, OUT_F), jnp.float32),
    )(x2d, mt, bias)


def kernel(x, conn, weight, bias_param):
    conn_i = conn.astype(jnp.int32)
    zeros_blk = jnp.zeros((SUB, IN_F), jnp.float32)
    mt = _build_mt(conn_i, weight, zeros_blk)
    return _matmul(x[0], mt, bias_param)


# speedup vs baseline: 1.1346x; 1.1346x over previous
import functools

import jax
import jax.numpy as jnp
from jax import lax
from jax.experimental import pallas as pl
from jax.experimental.pallas import tpu as pltpu
from jax.experimental.pallas import tpu_sc as plsc

IN_F = 2048
OUT_F = 2048
NCONN = 16

NUM_CORES = 2
NUM_SUBCORES = 16
NW = NUM_CORES * NUM_SUBCORES
ROWS_PER_W = OUT_F // NW
SUB = 2
NCHUNK = ROWS_PER_W // SUB
NBUF = 2


def _build_mt(conn_i32, weight, zeros_blk):
    mesh = plsc.VectorSubcoreMesh(core_axis_name="c", subcore_axis_name="s")

    @functools.partial(
        pl.kernel,
        mesh=mesh,
        out_type=jax.ShapeDtypeStruct((OUT_F, IN_F), jnp.float32),
        scratch_types=(
            [pltpu.VMEM((SUB, IN_F), jnp.float32) for _ in range(NBUF)]
            + [pltpu.VMEM((ROWS_PER_W * NCONN,), jnp.int32),
               pltpu.VMEM((ROWS_PER_W, NCONN), jnp.float32)]
            + [pltpu.SemaphoreType.DMA for _ in range(2 * NBUF)]
        ),
        compiler_params=pltpu.CompilerParams(needs_layout_passes=False),
    )
    def k(conn_hbm, w_hbm, z_hbm, mt_hbm, *refs):
        blks = refs[:NBUF]
        idx_v, wv = refs[NBUF], refs[NBUF + 1]
        sems = refs[NBUF + 2:NBUF + 2 + NBUF]
        zsems = refs[NBUF + 2 + NBUF:]
        wid = lax.axis_index("s") * NUM_CORES + lax.axis_index("c")
        base = wid * ROWS_PER_W
        zc = [pltpu.async_copy(z_hbm, blks[b], zsems[b])
              for b in range(NBUF)]
        pltpu.sync_copy(conn_hbm.at[pl.ds(base * NCONN, ROWS_PER_W * NCONN)],
                        idx_v)
        pltpu.sync_copy(w_hbm.at[pl.ds(base, ROWS_PER_W)], wv)
        for h in zc:
            h.wait()
        zvec = jnp.zeros((NCONN,), jnp.float32)

        def _chunk(g, _):
            for b in range(NBUF):
                c = g * NBUF + b
                blk = blks[b]

                @pl.when(g > 0)
                def _wait_and_restore(blk=blk, c=c, b=b):
                    pltpu.make_async_copy(
                        blk, mt_hbm.at[pl.ds(base + (c - NBUF) * SUB, SUB)],
                        sems[b]).wait()

                    def _restore(r, _):
                        o = ((c - NBUF) * SUB + r) * NCONN
                        rvec = jnp.full((NCONN,), r, jnp.int32)
                        plsc.store_scatter(blk,
                                           [rvec, idx_v[pl.ds(o, NCONN)]],
                                           zvec)
                        return _

                    lax.fori_loop(0, SUB, _restore, None, unroll=SUB)

                def _scatter(r, _, blk=blk, c=c):
                    o = (c * SUB + r) * NCONN
                    idx = idx_v[pl.ds(o, NCONN)]
                    w = wv[c * SUB + r]
                    rvec = jnp.full((NCONN,), r, jnp.int32)
                    plsc.addupdate_scatter(blk, [rvec, idx], w)
                    return _

                lax.fori_loop(0, SUB, _scatter, None, unroll=SUB)
                pltpu.async_copy(
                    blk, mt_hbm.at[pl.ds(base + c * SUB, SUB)], sems[b])
            return _

        lax.fori_loop(0, NCHUNK // NBUF, _chunk, None)
        for b in range(NBUF):
            c_last = NCHUNK - NBUF + b
            pltpu.make_async_copy(
                blks[b], mt_hbm.at[pl.ds(base + c_last * SUB, SUB)],
                sems[b]).wait()

    return k(conn_i32, weight, zeros_blk)


def _matmul(x2d, mt, bias):
    A_BLK = 256

    def body(x_ref, mt_ref, b_ref, o_ref):
        acc = lax.dot_general(
            x_ref[...], mt_ref[...],
            (((1,), (1,)), ((), ())),
            preferred_element_type=jnp.float32,
        )
        o_ref[...] = (acc + b_ref[...])[None]

    return pl.pallas_call(
        body,
        grid=(OUT_F // A_BLK,),
        in_specs=[
            pl.BlockSpec((2048, IN_F), lambda i: (0, 0)),
            pl.BlockSpec((A_BLK, IN_F), lambda i: (i, 0)),
            pl.BlockSpec((A_BLK,), lambda i: (i,)),
        ],
        out_specs=pl.BlockSpec((1, 2048, A_BLK), lambda i: (0, 0, i)),
        out_shape=jax.ShapeDtypeStruct((1, 2048, OUT_F), jnp.float32),
    )(x2d, mt, bias)


def kernel(x, conn, weight, bias_param):
    conn_i = conn.astype(jnp.int32)
    zeros_blk = jnp.zeros((SUB, IN_F), jnp.float32)
    mt = _build_mt(conn_i, weight, zeros_blk)
    return _matmul(x[0], mt, bias_param)
